# Initial kernel scaffold; baseline (speedup 1.0000x reference)
import jax
import jax.numpy as jnp
from jax.experimental import pallas as pl


def kernel(x, edge_index, edge_type, features, batch, W_enc, b_enc, Ws1, Wr1, b1, Ws2, Wr2, b2, a_prelu, W_fc1, b_fc1, W_fc2, b_fc2, W_out, b_out):
    raise NotImplementedError("write your pallas kernel here")


# speedup vs baseline: 4.4006x; 4.4006x over previous
import functools

import jax
import jax.numpy as jnp
from jax import lax
from jax.experimental import pallas as pl
from jax.experimental.pallas import tpu as pltpu
from jax.experimental.pallas import tpu_sc as plsc

N = 10000
NP = 10240
E = 320000
G = 64
D = 128
R = 6

NC = 2
NS = 16
NW = NC * NS
ROWS_PER_TILE = NP // NS

_MESH = plsc.VectorSubcoreMesh(core_axis_name="c", subcore_axis_name="s",
                               num_cores=NC, num_subcores=NS)


_CNT_CHUNK = 16
_CNT_CHUNKS_PER_W = E // _CNT_CHUNK // NW


@functools.partial(
    pl.kernel,
    out_type=jax.ShapeDtypeStruct((2, NP * 16), jnp.float32),
    mesh=_MESH,
    scratch_types=[
        pltpu.VMEM((_CNT_CHUNK,), jnp.int32),
        pltpu.VMEM((_CNT_CHUNK,), jnp.int32),
        pltpu.VMEM((_CNT_CHUNK,), jnp.int32),
        pltpu.VMEM((_CNT_CHUNK,), jnp.float32),
        pltpu.VMEM_SHARED((NP * 16,), jnp.float32),
    ],
)
def _count_pass(dst_hbm, ty_hbm, zflat_hbm, cnt_out, dbuf, tbuf, fbuf, ones_v,
                shared_cnt):
    cid = lax.axis_index("c")
    sid = lax.axis_index("s")
    wid = sid * NC + cid
    span = ROWS_PER_TILE * 16

    pltpu.sync_copy(zflat_hbm.at[pl.ds(sid * span, span)],
                    shared_cnt.at[pl.ds(sid * span, span)])
    ones_v[...] = jnp.zeros((16,), jnp.float32) + 1.0
    plsc.subcore_barrier()

    @pl.loop(0, _CNT_CHUNKS_PER_W)
    def _(i):
        base = (i * NW + wid) * _CNT_CHUNK
        pltpu.sync_copy(dst_hbm.at[pl.ds(base, _CNT_CHUNK)], dbuf)
        pltpu.sync_copy(ty_hbm.at[pl.ds(base, _CNT_CHUNK)], tbuf)
        fbuf[...] = dbuf[...] * 16 + tbuf[...]
        pltpu.sync_copy(ones_v, shared_cnt.at[fbuf], add=True)

    plsc.subcore_barrier()
    pltpu.sync_copy(shared_cnt.at[pl.ds(sid * span, span)],
                    cnt_out.at[cid, pl.ds(sid * span, span)])


@functools.partial(
    pl.kernel,
    out_type=(
        jax.ShapeDtypeStruct((E,), jnp.float32),
        jax.ShapeDtypeStruct((E,), jnp.int32),
    ),
    mesh=_MESH,
    scratch_types=[
        pltpu.VMEM((_CNT_CHUNK,), jnp.int32),
        pltpu.VMEM((_CNT_CHUNK,), jnp.int32),
        pltpu.VMEM((_CNT_CHUNK,), jnp.int32),
        pltpu.VMEM((_CNT_CHUNK,), jnp.int32),
        pltpu.VMEM((_CNT_CHUNK,), jnp.float32),
        pltpu.VMEM((_CNT_CHUNK,), jnp.float32),
        pltpu.VMEM((_CNT_CHUNK,), jnp.float32),
        pltpu.VMEM((_CNT_CHUNK,), jnp.int32),
        pltpu.SemaphoreType.DMA,
        pltpu.SemaphoreType.DMA,
    ],
)
def _scale_pass(src_hbm, dst_hbm, ty_hbm, cntA_hbm, cntB_hbm, inv_out, gidx_out,
                sbuf, dbuf, tbuf, fbuf, cA, cB, ivbuf, gbuf, semA, semB):
    cid = lax.axis_index("c")
    sid = lax.axis_index("s")
    wid = sid * NC + cid

    @pl.loop(0, _CNT_CHUNKS_PER_W)
    def _(i):
        base = (i * NW + wid) * _CNT_CHUNK
        pltpu.sync_copy(dst_hbm.at[pl.ds(base, _CNT_CHUNK)], dbuf)
        pltpu.sync_copy(src_hbm.at[pl.ds(base, _CNT_CHUNK)], sbuf)
        pltpu.sync_copy(ty_hbm.at[pl.ds(base, _CNT_CHUNK)], tbuf)
        ty = tbuf[...]
        fbuf[...] = dbuf[...] * 16 + ty
        cpA = pltpu.async_copy(cntA_hbm.at[fbuf], cA, semA)
        cpB = pltpu.async_copy(cntB_hbm.at[fbuf], cB, semB)
        cpA.wait()
        cpB.wait()
        c = cA[...] + cB[...]
        ivbuf[...] = 1.0 / jnp.maximum(c, 1.0)
        gbuf[...] = ty * N + sbuf[...]
        pltpu.sync_copy(ivbuf, inv_out.at[pl.ds(base, _CNT_CHUNK)])
        pltpu.sync_copy(gbuf, gidx_out.at[pl.ds(base, _CNT_CHUNK)])


_MSG_CHUNK = 64
_MSG_CHUNKS = E // _MSG_CHUNK
_MSG_ITERS = -(-_MSG_CHUNKS // NW)


@functools.partial(
    pl.kernel,
    out_type=jax.ShapeDtypeStruct((2, NP, D), jnp.float32),
    mesh=_MESH,
    scratch_types=[
        pltpu.VMEM((_MSG_CHUNK,), jnp.int32),
        pltpu.VMEM((_MSG_CHUNK,), jnp.int32),
        pltpu.VMEM((_MSG_CHUNK,), jnp.float32),
        pltpu.VMEM((_MSG_CHUNK, D), jnp.float32),
        pltpu.VMEM_SHARED((NP, D), jnp.float32),
        pltpu.SemaphoreType.DMA,
    ],
)
def _message_pass(hcat_hbm, gidx_hbm, inv_hbm, dst_hbm, zD_hbm, part_out,
                  gbuf, dbuf, ivbuf, rows, shared_acc, sem):
    cid = lax.axis_index("c")
    sid = lax.axis_index("s")
    wid = sid * NC + cid

    pltpu.sync_copy(zD_hbm.at[pl.ds(sid * ROWS_PER_TILE, ROWS_PER_TILE)],
                    shared_acc.at[pl.ds(sid * ROWS_PER_TILE, ROWS_PER_TILE)])
    plsc.subcore_barrier()

    @pl.loop(0, _MSG_ITERS)
    def _(i):
        chunk = i * NW + wid

        @pl.when(chunk < _MSG_CHUNKS)
        def _():
            base = chunk * _MSG_CHUNK
            pltpu.sync_copy(gidx_hbm.at[pl.ds(base, _MSG_CHUNK)], gbuf)
            pltpu.async_copy(hcat_hbm.at[gbuf], rows, sem).wait()
            pltpu.sync_copy(inv_hbm.at[pl.ds(base, _MSG_CHUNK)], ivbuf)
            pltpu.sync_copy(dst_hbm.at[pl.ds(base, _MSG_CHUNK)], dbuf)

            @pl.loop(0, _MSG_CHUNK // 16)
            def _(g):
                iv16 = ivbuf[pl.ds(g * 16, 16)]
                for k in range(16):
                    s = iv16[k]
                    j = g * 16 + k
                    for q in range(D // 16):
                        rows[j, pl.ds(q * 16, 16)] = rows[j, pl.ds(q * 16, 16)] * s

            pltpu.sync_copy(rows, shared_acc.at[dbuf], add=True)

    plsc.subcore_barrier()
    pltpu.sync_copy(shared_acc.at[pl.ds(sid * ROWS_PER_TILE, ROWS_PER_TILE)],
                    part_out.at[cid, pl.ds(sid * ROWS_PER_TILE, ROWS_PER_TILE)])


_BN = 1000
_GRID = N // _BN


def _tc_enc_body(x_ref, we_ref, be_ref, ws_ref, h_ref, hcat_ref):
    h = jnp.dot(x_ref[...], we_ref[...], preferred_element_type=jnp.float32)
    h = h + be_ref[...]
    h_ref[...] = h
    for r in range(R):
        hcat_ref[r] = jnp.dot(h, ws_ref[r], preferred_element_type=jnp.float32)


_tc_enc = pl.pallas_call(
    _tc_enc_body,
    grid=(_GRID,),
    in_specs=[
        pl.BlockSpec((_BN, 13), lambda i: (i, 0)),
        pl.BlockSpec((13, D), lambda i: (0, 0)),
        pl.BlockSpec((1, D), lambda i: (0, 0)),
        pl.BlockSpec((R, D, D), lambda i: (0, 0, 0)),
    ],
    out_specs=[
        pl.BlockSpec((_BN, D), lambda i: (i, 0)),
        pl.BlockSpec((R, _BN, D), lambda i: (0, i, 0)),
    ],
    out_shape=[
        jax.ShapeDtypeStruct((N, D), jnp.float32),
        jax.ShapeDtypeStruct((R, N, D), jnp.float32),
    ],
)


def _layer_tail(t, a):
    t = jnp.where(t >= 0.0, t, a * t)
    nrm = jnp.sqrt(jnp.sum(t * t, axis=1, keepdims=True))
    return t / jnp.maximum(nrm, 1e-12)


def _tc_combine_body(p_ref, h_ref, wr_ref, b_ref, a_ref, ws_ref, h2_ref, hcat_ref):
    h = h_ref[...]
    t = p_ref[0] + p_ref[1] + b_ref[...] + h
    t = t + jnp.dot(h, wr_ref[...], preferred_element_type=jnp.float32)
    h2 = _layer_tail(t, a_ref[0, 0])
    h2_ref[...] = h2
    for r in range(R):
        hcat_ref[r] = jnp.dot(h2, ws_ref[r], preferred_element_type=jnp.float32)


_tc_combine = pl.pallas_call(
    _tc_combine_body,
    grid=(_GRID,),
    in_specs=[
        pl.BlockSpec((2, _BN, D), lambda i: (0, i, 0)),
        pl.BlockSpec((_BN, D), lambda i: (i, 0)),
        pl.BlockSpec((D, D), lambda i: (0, 0)),
        pl.BlockSpec((1, D), lambda i: (0, 0)),
        pl.BlockSpec((1, 1), lambda i: (0, 0)),
        pl.BlockSpec((R, D, D), lambda i: (0, 0, 0)),
    ],
    out_specs=[
        pl.BlockSpec((_BN, D), lambda i: (i, 0)),
        pl.BlockSpec((R, _BN, D), lambda i: (0, i, 0)),
    ],
    out_shape=[
        jax.ShapeDtypeStruct((N, D), jnp.float32),
        jax.ShapeDtypeStruct((R, N, D), jnp.float32),
    ],
)


def _tc_pool_body(p_ref, h_ref, wr_ref, b_ref, a_ref, batch_ref, ps_ref, pc_ref):
    i = pl.program_id(0)
    h = h_ref[...]
    t = p_ref[0] + p_ref[1] + b_ref[...] + h
    t = t + jnp.dot(h, wr_ref[...], preferred_element_type=jnp.float32)
    h3 = _layer_tail(t, a_ref[0, 0])
    onehot = (batch_ref[...] == lax.broadcasted_iota(jnp.int32, (1, G), 1))
    onehot = onehot.astype(jnp.float32)
    ps = lax.dot_general(onehot, h3, (((0,), (0,)), ((), ())),
                         preferred_element_type=jnp.float32)
    pc = jnp.sum(onehot, axis=0, keepdims=True)

    @pl.when(i == 0)
    def _():
        ps_ref[...] = jnp.zeros_like(ps_ref)
        pc_ref[...] = jnp.zeros_like(pc_ref)

    ps_ref[...] += ps
    pc_ref[...] += pc


_tc_pool = pl.pallas_call(
    _tc_pool_body,
    grid=(_GRID,),
    in_specs=[
        pl.BlockSpec((2, _BN, D), lambda i: (0, i, 0)),
        pl.BlockSpec((_BN, D), lambda i: (i, 0)),
        pl.BlockSpec((D, D), lambda i: (0, 0)),
        pl.BlockSpec((1, D), lambda i: (0, 0)),
        pl.BlockSpec((1, 1), lambda i: (0, 0)),
        pl.BlockSpec((_BN, 1), lambda i: (i, 0)),
    ],
    out_specs=[
        pl.BlockSpec((G, D), lambda i: (0, 0)),
        pl.BlockSpec((1, G), lambda i: (0, 0)),
    ],
    out_shape=[
        jax.ShapeDtypeStruct((G, D), jnp.float32),
        jax.ShapeDtypeStruct((1, G), jnp.float32),
    ],
)


def _tc_mlp_body(ps_ref, pc_ref, feat_ref, w1_ref, b1_ref, w2_ref, b2_ref,
                 wo_ref, bo_ref, out_ref):
    cnt = jnp.maximum(pc_ref[...], 1.0)
    gm = ps_ref[...] * (1.0 / jnp.transpose(cnt))
    g = jnp.concatenate([gm, feat_ref[...]], axis=1)
    z = jnp.dot(g, w1_ref[...], preferred_element_type=jnp.float32) + b1_ref[...]
    z = jnp.maximum(z, 0.0)
    z = jnp.dot(z, w2_ref[...], preferred_element_type=jnp.float32) + b2_ref[...]
    z = jnp.maximum(z, 0.0)
    out_ref[...] = jnp.dot(z, wo_ref[...], preferred_element_type=jnp.float32) + bo_ref[...]


_tc_mlp = pl.pallas_call(
    _tc_mlp_body,
    out_shape=jax.ShapeDtypeStruct((G, 1), jnp.float32),
)


def kernel(x, edge_index, edge_type, features, batch,
           W_enc, b_enc, Ws1, Wr1, b1, Ws2, Wr2, b2, a_prelu,
           W_fc1, b_fc1, W_fc2, b_fc2, W_out, b_out):
    src = edge_index[0]
    dst = edge_index[1]
    zflat = jnp.zeros((NP * 16,), jnp.float32)
    zD = jnp.zeros((NP, D), jnp.float32)
    a2 = a_prelu.reshape(1, 1)

    cnt = _count_pass(dst, edge_type, zflat)
    inv_s, gidx = _scale_pass(src, dst, edge_type, cnt[0], cnt[1])

    h, hcat1 = _tc_enc(x, W_enc, b_enc.reshape(1, D), Ws1)
    p1 = _message_pass(hcat1.reshape(R * N, D), gidx, inv_s, dst, zD)
    h2, hcat2 = _tc_combine(p1, h, Wr1, b1.reshape(1, D), a2, Ws2)
    p2 = _message_pass(hcat2.reshape(R * N, D), gidx, inv_s, dst, zD)
    ps, pc = _tc_pool(p2, h2, Wr2, b2.reshape(1, D), a2, batch.reshape(N, 1))
    out = _tc_mlp(ps, pc, features, W_fc1, b_fc1.reshape(1, 256), W_fc2,
                  b_fc2.reshape(1, 128), W_out, b_out.reshape(1, 1))
    return out


# speedup vs baseline: 13.8160x; 3.1396x over previous
import functools

import jax
import jax.numpy as jnp
from jax import lax
from jax.experimental import pallas as pl
from jax.experimental.pallas import tpu as pltpu
from jax.experimental.pallas import tpu_sc as plsc

N = 10000
NP = 10240
E = 320000
G = 64
D = 128
R = 6

NC = 2
NS = 16
NW = NC * NS
ROWS_PER_TILE = NP // NS

_MESH = plsc.VectorSubcoreMesh(core_axis_name="c", subcore_axis_name="s",
                               num_cores=NC, num_subcores=NS)


_ECH = 128
_ECHUNKS = E // _ECH
_EITERS = -(-_ECHUNKS // NW)
_CSTRIDE = 8
_CSPAN = NP * _CSTRIDE // NS


@functools.partial(
    pl.kernel,
    out_type=jax.ShapeDtypeStruct((2, NP * _CSTRIDE), jnp.float32),
    mesh=_MESH,
    scratch_types=[
        pltpu.VMEM((_ECH,), jnp.int32),
        pltpu.VMEM((_ECH,), jnp.int32),
        pltpu.VMEM((_ECH,), jnp.int32),
        pltpu.VMEM((_ECH,), jnp.float32),
        pltpu.VMEM_SHARED((NP * _CSTRIDE,), jnp.float32),
    ],
)
def _count_pass(dst_hbm, ty_hbm, zflat_hbm, cnt_out, dbuf, tbuf, fbuf, ones_v,
                shared_cnt):
    cid = lax.axis_index("c")
    sid = lax.axis_index("s")
    wid = sid * NC + cid

    pltpu.sync_copy(zflat_hbm.at[pl.ds(sid * _CSPAN, _CSPAN)],
                    shared_cnt.at[pl.ds(sid * _CSPAN, _CSPAN)])
    one = jnp.zeros((16,), jnp.float32) + 1.0
    for m in range(_ECH // 16):
        ones_v[pl.ds(m * 16, 16)] = one
    plsc.subcore_barrier()

    @pl.loop(0, _EITERS)
    def _(i):
        chunk = i * NW + wid

        @pl.when(chunk < _ECHUNKS)
        def _():
            base = chunk * _ECH
            pltpu.sync_copy(dst_hbm.at[pl.ds(base, _ECH)], dbuf)
            pltpu.sync_copy(ty_hbm.at[pl.ds(base, _ECH)], tbuf)

            @pl.loop(0, _ECH // 16)
            def _(m):
                sl = pl.ds(m * 16, 16)
                fbuf[sl] = dbuf[sl] * _CSTRIDE + tbuf[sl]

            pltpu.sync_copy(ones_v, shared_cnt.at[fbuf], add=True)

    plsc.subcore_barrier()
    pltpu.sync_copy(shared_cnt.at[pl.ds(sid * _CSPAN, _CSPAN)],
                    cnt_out.at[cid, pl.ds(sid * _CSPAN, _CSPAN)])


def _tc_inv_body(c_ref, inv_ref):
    inv_ref[...] = 1.0 / jnp.maximum(c_ref[0] + c_ref[1], 1.0)


_tc_inv = pl.pallas_call(
    _tc_inv_body,
    out_shape=jax.ShapeDtypeStruct((NP * _CSTRIDE // 128, 128), jnp.float32),
)


@functools.partial(
    pl.kernel,
    out_type=(
        jax.ShapeDtypeStruct((E,), jnp.float32),
        jax.ShapeDtypeStruct((E,), jnp.int32),
    ),
    mesh=_MESH,
    scratch_types=[
        pltpu.VMEM((_ECH,), jnp.int32),
        pltpu.VMEM((_ECH,), jnp.int32),
        pltpu.VMEM((_ECH,), jnp.int32),
        pltpu.VMEM((_ECH,), jnp.int32),
        pltpu.VMEM((_ECH,), jnp.float32),
        pltpu.VMEM((_ECH,), jnp.int32),
        pltpu.SemaphoreType.DMA,
    ],
)
def _scale_pass(src_hbm, dst_hbm, ty_hbm, invtab_hbm, inv_out, gidx_out,
                sbuf, dbuf, tbuf, fbuf, cbuf, gbuf, sem):
    cid = lax.axis_index("c")
    sid = lax.axis_index("s")
    wid = sid * NC + cid

    @pl.loop(0, _EITERS)
    def _(i):
        chunk = i * NW + wid

        @pl.when(chunk < _ECHUNKS)
        def _():
            base = chunk * _ECH
            pltpu.sync_copy(dst_hbm.at[pl.ds(base, _ECH)], dbuf)
            pltpu.sync_copy(src_hbm.at[pl.ds(base, _ECH)], sbuf)
            pltpu.sync_copy(ty_hbm.at[pl.ds(base, _ECH)], tbuf)

            @pl.loop(0, _ECH // 16)
            def _(m):
                sl = pl.ds(m * 16, 16)
                fbuf[sl] = dbuf[sl] * _CSTRIDE + tbuf[sl]
                gbuf[sl] = tbuf[sl] * N + sbuf[sl]

            pltpu.async_copy(invtab_hbm.at[fbuf], cbuf, sem).wait()
            pltpu.sync_copy(cbuf, inv_out.at[pl.ds(base, _ECH)])
            pltpu.sync_copy(gbuf, gidx_out.at[pl.ds(base, _ECH)])


_MB = 128
_MCHUNKS = E // _MB
_MITERS = 79


@functools.partial(
    pl.kernel,
    out_type=jax.ShapeDtypeStruct((2, NP, D), jnp.float32),
    mesh=_MESH,
    scratch_types=[
        pltpu.VMEM((_MB,), jnp.int32),
        pltpu.VMEM((_MB,), jnp.int32),
        pltpu.VMEM((_MB,), jnp.float32),
        pltpu.VMEM((_MB, D), jnp.float32),
        pltpu.VMEM_SHARED((NP, D), jnp.float32),
        pltpu.SemaphoreType.DMA,
    ],
)
def _message_pass(hcat_hbm, gidx_hbm, inv_hbm, dst_hbm, zD_hbm, part_out,
                  gbuf, dbuf, ivbuf, rows, shared_acc, semg):
    cid = lax.axis_index("c")
    sid = lax.axis_index("s")
    wid = sid * NC + cid

    pltpu.sync_copy(zD_hbm.at[pl.ds(sid * ROWS_PER_TILE, ROWS_PER_TILE)],
                    shared_acc.at[pl.ds(sid * ROWS_PER_TILE, ROWS_PER_TILE)])
    plsc.subcore_barrier()

    @pl.loop(0, _MITERS)
    def _(i):
        chunk = i * NW + wid

        @pl.when(chunk < _MCHUNKS)
        def _():
            base = chunk * _MB
            pltpu.sync_copy(gidx_hbm.at[pl.ds(base, _MB)], gbuf)
            pltpu.async_copy(hcat_hbm.at[gbuf], rows, semg)
            pltpu.sync_copy(inv_hbm.at[pl.ds(base, _MB)], ivbuf)
            pltpu.sync_copy(dst_hbm.at[pl.ds(base, _MB)], dbuf)
            pltpu.make_async_copy(hcat_hbm.at[gbuf], rows, semg).wait()

            @pl.loop(0, _MB // 16)
            def _(g):
                iv16 = ivbuf[pl.ds(g * 16, 16)]
                for k in range(16):
                    sc = iv16[k]
                    j = g * 16 + k
                    for q in range(D // 16):
                        rows[j, pl.ds(q * 16, 16)] = rows[j, pl.ds(q * 16, 16)] * sc

            pltpu.sync_copy(rows, shared_acc.at[dbuf], add=True)

    plsc.subcore_barrier()
    pltpu.sync_copy(shared_acc.at[pl.ds(sid * ROWS_PER_TILE, ROWS_PER_TILE)],
                    part_out.at[cid, pl.ds(sid * ROWS_PER_TILE, ROWS_PER_TILE)])


_BN = 1000
_GRID = N // _BN


def _tc_enc_body(x_ref, we_ref, be_ref, ws_ref, h_ref, hcat_ref):
    h = jnp.dot(x_ref[...], we_ref[...], preferred_element_type=jnp.float32)
    h = h + be_ref[...]
    h_ref[...] = h
    for r in range(R):
        hcat_ref[r] = jnp.dot(h, ws_ref[r], preferred_element_type=jnp.float32)


_tc_enc = pl.pallas_call(
    _tc_enc_body,
    grid=(_GRID,),
    in_specs=[
        pl.BlockSpec((_BN, 13), lambda i: (i, 0)),
        pl.BlockSpec((13, D), lambda i: (0, 0)),
        pl.BlockSpec((1, D), lambda i: (0, 0)),
        pl.BlockSpec((R, D, D), lambda i: (0, 0, 0)),
    ],
    out_specs=[
        pl.BlockSpec((_BN, D), lambda i: (i, 0)),
        pl.BlockSpec((R, _BN, D), lambda i: (0, i, 0)),
    ],
    out_shape=[
        jax.ShapeDtypeStruct((N, D), jnp.float32),
        jax.ShapeDtypeStruct((R, N, D), jnp.float32),
    ],
)


def _layer_tail(t, a):
    t = jnp.where(t >= 0.0, t, a * t)
    nrm = jnp.sqrt(jnp.sum(t * t, axis=1, keepdims=True))
    return t / jnp.maximum(nrm, 1e-12)


def _tc_combine_body(p_ref, h_ref, wr_ref, b_ref, a_ref, ws_ref, h2_ref, hcat_ref):
    h = h_ref[...]
    t = p_ref[0] + p_ref[1] + b_ref[...] + h
    t = t + jnp.dot(h, wr_ref[...], preferred_element_type=jnp.float32)
    h2 = _layer_tail(t, a_ref[0, 0])
    h2_ref[...] = h2
    for r in range(R):
        hcat_ref[r] = jnp.dot(h2, ws_ref[r], preferred_element_type=jnp.float32)


_tc_combine = pl.pallas_call(
    _tc_combine_body,
    grid=(_GRID,),
    in_specs=[
        pl.BlockSpec((2, _BN, D), lambda i: (0, i, 0)),
        pl.BlockSpec((_BN, D), lambda i: (i, 0)),
        pl.BlockSpec((D, D), lambda i: (0, 0)),
        pl.BlockSpec((1, D), lambda i: (0, 0)),
        pl.BlockSpec((1, 1), lambda i: (0, 0)),
        pl.BlockSpec((R, D, D), lambda i: (0, 0, 0)),
    ],
    out_specs=[
        pl.BlockSpec((_BN, D), lambda i: (i, 0)),
        pl.BlockSpec((R, _BN, D), lambda i: (0, i, 0)),
    ],
    out_shape=[
        jax.ShapeDtypeStruct((N, D), jnp.float32),
        jax.ShapeDtypeStruct((R, N, D), jnp.float32),
    ],
)


def _tc_pool_body(p_ref, h_ref, wr_ref, b_ref, a_ref, batch_ref, ps_ref, pc_ref):
    i = pl.program_id(0)
    h = h_ref[...]
    t = p_ref[0] + p_ref[1] + b_ref[...] + h
    t = t + jnp.dot(h, wr_ref[...], preferred_element_type=jnp.float32)
    h3 = _layer_tail(t, a_ref[0, 0])
    onehot = (batch_ref[...] == lax.broadcasted_iota(jnp.int32, (1, G), 1))
    onehot = onehot.astype(jnp.float32)
    ps = lax.dot_general(onehot, h3, (((0,), (0,)), ((), ())),
                         preferred_element_type=jnp.float32)
    pc = jnp.sum(onehot, axis=0, keepdims=True)

    @pl.when(i == 0)
    def _():
        ps_ref[...] = jnp.zeros_like(ps_ref)
        pc_ref[...] = jnp.zeros_like(pc_ref)

    ps_ref[...] += ps
    pc_ref[...] += pc


_tc_pool = pl.pallas_call(
    _tc_pool_body,
    grid=(_GRID,),
    in_specs=[
        pl.BlockSpec((2, _BN, D), lambda i: (0, i, 0)),
        pl.BlockSpec((_BN, D), lambda i: (i, 0)),
        pl.BlockSpec((D, D), lambda i: (0, 0)),
        pl.BlockSpec((1, D), lambda i: (0, 0)),
        pl.BlockSpec((1, 1), lambda i: (0, 0)),
        pl.BlockSpec((_BN, 1), lambda i: (i, 0)),
    ],
    out_specs=[
        pl.BlockSpec((G, D), lambda i: (0, 0)),
        pl.BlockSpec((1, G), lambda i: (0, 0)),
    ],
    out_shape=[
        jax.ShapeDtypeStruct((G, D), jnp.float32),
        jax.ShapeDtypeStruct((1, G), jnp.float32),
    ],
)


def _tc_mlp_body(ps_ref, pc_ref, feat_ref, w1_ref, b1_ref, w2_ref, b2_ref,
                 wo_ref, bo_ref, out_ref):
    cnt = jnp.maximum(pc_ref[...], 1.0)
    gm = ps_ref[...] * (1.0 / jnp.transpose(cnt))
    g = jnp.concatenate([gm, feat_ref[...]], axis=1)
    z = jnp.dot(g, w1_ref[...], preferred_element_type=jnp.float32) + b1_ref[...]
    z = jnp.maximum(z, 0.0)
    z = jnp.dot(z, w2_ref[...], preferred_element_type=jnp.float32) + b2_ref[...]
    z = jnp.maximum(z, 0.0)
    out_ref[...] = jnp.dot(z, wo_ref[...], preferred_element_type=jnp.float32) + bo_ref[...]


_tc_mlp = pl.pallas_call(
    _tc_mlp_body,
    out_shape=jax.ShapeDtypeStruct((G, 1), jnp.float32),
)


def kernel(x, edge_index, edge_type, features, batch,
           W_enc, b_enc, Ws1, Wr1, b1, Ws2, Wr2, b2, a_prelu,
           W_fc1, b_fc1, W_fc2, b_fc2, W_out, b_out):
    src = edge_index[0]
    dst = edge_index[1]
    zflat = jnp.zeros((NP * _CSTRIDE,), jnp.float32)
    zD = jnp.zeros((NP, D), jnp.float32)
    a2 = a_prelu.reshape(1, 1)

    cnt = _count_pass(dst, edge_type, zflat)
    invtab = _tc_inv(cnt.reshape(2, NP * _CSTRIDE // 128, 128))
    inv_s, gidx = _scale_pass(src, dst, edge_type, invtab.reshape(NP * _CSTRIDE))

    h, hcat1 = _tc_enc(x, W_enc, b_enc.reshape(1, D), Ws1)
    p1 = _message_pass(hcat1.reshape(R * N, D), gidx, inv_s, dst, zD)
    h2, hcat2 = _tc_combine(p1, h, Wr1, b1.reshape(1, D), a2, Ws2)
    p2 = _message_pass(hcat2.reshape(R * N, D), gidx, inv_s, dst, zD)
    ps, pc = _tc_pool(p2, h2, Wr2, b2.reshape(1, D), a2, batch.reshape(N, 1))
    out = _tc_mlp(ps, pc, features, W_fc1, b_fc1.reshape(1, 256), W_fc2,
                  b_fc2.reshape(1, 128), W_out, b_out.reshape(1, 1))
    return out
